# baseline (speedup 1.0000x reference)
import functools
import math

import jax
import jax.numpy as jnp
from jax import lax
from jax.experimental import pallas as pl
from jax.experimental.pallas import tpu as pltpu
from jax.experimental.pallas import tpu_sc as plsc

N = 10000
E = 320000
D = 128
NC = 2
NS = 16
NW = NC * NS
EPW = E // NW
CH = 80
NCHUNK = EPW // CH
NCHP = 128
NPAD = 10240
SSL = NPAD // NS
INV_SQRT_D = 1.0 / math.sqrt(float(D))

_mesh = plsc.VectorSubcoreMesh(core_axis_name="c", subcore_axis_name="s")
_GD = lax.GatherDimensionNumbers(
    offset_dims=(), collapsed_slice_dims=(0,), start_index_map=(0,))


def _lane_shuffle(v, idx):
    return lax.gather(v, idx.reshape(16, 1), dimension_numbers=_GD,
                      slice_sizes=(1,), mode=lax.GatherScatterMode.PROMISE_IN_BOUNDS)


@functools.partial(
    pl.kernel,
    out_type=(
        jax.ShapeDtypeStruct((NW, NCHP, CH), jnp.float32),
        jax.ShapeDtypeStruct((NPAD,), jnp.float32),
        jax.ShapeDtypeStruct((NPAD,), jnp.float32),
    ),
    mesh=_mesh,
    scratch_types=[
        pltpu.VMEM((NCHP, CH), jnp.int32),
        pltpu.VMEM((NCHP, CH), jnp.int32),
        pltpu.VMEM((NCHP, CH), jnp.float32),
        pltpu.VMEM((CH, D), jnp.float32),
        pltpu.VMEM((CH, D), jnp.float32),
        pltpu.VMEM((CH, D), jnp.float32),
        pltpu.VMEM((CH, D), jnp.float32),
        pltpu.VMEM((256,), jnp.float32),
        pltpu.VMEM((SSL,), jnp.float32),
        pltpu.VMEM_SHARED((NPAD,), jnp.float32),
        pltpu.SemaphoreType.DMA,
        pltpu.SemaphoreType.DMA,
        pltpu.SemaphoreType.DMA,
        pltpu.SemaphoreType.DMA,
    ],
)
def _sc_edge_logits(q_hbm, k_hbm, src_hbm, dst_hbm, a_hbm, s0_hbm, s1_hbm,
                    src_v, dst_v, a_v, qra, kra, qrb, krb, dps, z_v, ssh,
                    sem_qa, sem_ka, sem_qb, sem_kb):
    cid = lax.axis_index("c")
    sid = lax.axis_index("s")
    wid = sid * NC + cid
    lane = lax.iota(jnp.int32, 16)

    def zbody(i, _):
        z_v[pl.ds(i * 16, 16)] = jnp.zeros((16,), jnp.float32)
        return 0
    lax.fori_loop(0, SSL // 16, zbody, 0)
    pltpu.sync_copy(z_v, ssh.at[pl.ds(sid * SSL, SSL)])
    pltpu.sync_copy(src_hbm.at[wid], src_v)
    pltpu.sync_copy(dst_hbm.at[wid], dst_v)
    plsc.subcore_barrier()

    def compute_chunk(j, qr, kr):
        for g in range(CH // 16):
            logits = jnp.zeros((16,), jnp.float32)
            for e in range(16):
                row = g * 16 + e

                def dbody(dd, acc):
                    ix = pl.ds(dd * 16, 16)
                    return acc + qr[row, ix] * kr[row, ix]
                acc = lax.fori_loop(0, D // 16, dbody,
                                    jnp.zeros((16,), jnp.float32), unroll=8)
                for sh in (8, 4, 2, 1):
                    acc = acc + _lane_shuffle(acc, jnp.bitwise_xor(lane, sh))
                logits = jnp.where(lane == e, acc, logits)
            a_v[j, pl.ds(g * 16, 16)] = jnp.exp(logits * INV_SQRT_D)
        pltpu.sync_copy(a_v.at[j], ssh.at[dst_v.at[j]], add=True)

    def pair_body(p, _):
        ja = 2 * p
        jb = ja + 1
        cqa = pltpu.async_copy(q_hbm.at[dst_v.at[ja]], qra, sem_qa)
        cka = pltpu.async_copy(k_hbm.at[src_v.at[ja]], kra, sem_ka)
        cqb = pltpu.async_copy(q_hbm.at[dst_v.at[jb]], qrb, sem_qb)
        ckb = pltpu.async_copy(k_hbm.at[src_v.at[jb]], krb, sem_kb)
        cqa.wait()
        cka.wait()
        compute_chunk(ja, qra, kra)
        cqb.wait()
        ckb.wait()
        compute_chunk(jb, qrb, krb)
        return 0

    lax.fori_loop(0, NCHUNK // 2, pair_body, 0)
    jt = NCHUNK - 1
    cq = pltpu.async_copy(q_hbm.at[dst_v.at[jt]], qra, sem_qa)
    ck = pltpu.async_copy(k_hbm.at[src_v.at[jt]], kra, sem_ka)
    cq.wait()
    ck.wait()
    compute_chunk(jt, qra, kra)
    pltpu.sync_copy(a_v, a_hbm.at[wid])
    plsc.subcore_barrier()
    pltpu.sync_copy(ssh.at[pl.ds(sid * SSL, SSL)], z_v)

    @pl.when(cid == 0)
    def _():
        pltpu.sync_copy(z_v, s0_hbm.at[pl.ds(sid * SSL, SSL)])

    @pl.when(cid == 1)
    def _():
        pltpu.sync_copy(z_v, s1_hbm.at[pl.ds(sid * SSL, SSL)])


NSEG = 2
SEGR = NPAD // NSEG
TRASH = SEGR
ASHR = SEGR + 16
RPS = SEGR // NS


@functools.partial(
    pl.kernel,
    out_type=jax.ShapeDtypeStruct((NC, NPAD, D), jnp.float32),
    mesh=_mesh,
    scratch_types=[
        pltpu.VMEM((NCHP, CH), jnp.int32),
        pltpu.VMEM((NCHP, CH), jnp.int32),
        pltpu.VMEM((NCHP, CH), jnp.float32),
        pltpu.VMEM((NPAD + 16,), jnp.float32),
        pltpu.VMEM((CH,), jnp.int32),
        pltpu.VMEM((CH, D), jnp.float32),
        pltpu.VMEM((CH, D), jnp.float32),
        pltpu.VMEM((16, D), jnp.float32),
        pltpu.VMEM_SHARED((ASHR, D), jnp.float32),
        pltpu.SemaphoreType.DMA,
        pltpu.SemaphoreType.DMA,
    ],
)
def _sc_aggregate(v_hbm, rs_hbm, src_hbm, dst_hbm, a_hbm, agg_hbm,
                  src_v, dst_v, a_v, rs_v, idx_c, vra, vrb, z_v, ash,
                  sem_va, sem_vb):
    cid = lax.axis_index("c")
    sid = lax.axis_index("s")
    wid = sid * NC + cid
    lane = lax.iota(jnp.int32, 16)

    def zbody(i, _):
        r = i // (D // 16)
        col = (i % (D // 16)) * 16
        z_v[r, pl.ds(col, 16)] = jnp.zeros((16,), jnp.float32)
        return 0
    lax.fori_loop(0, 16 * (D // 16), zbody, 0)

    pltpu.sync_copy(rs_hbm, rs_v.at[pl.ds(0, NPAD)])
    pltpu.sync_copy(src_hbm.at[wid], src_v)
    pltpu.sync_copy(dst_hbm.at[wid], dst_v)
    pltpu.sync_copy(a_hbm.at[wid], a_v)

    def alpha_body(j, _):
        for g in range(CH // 16):
            dst16 = dst_v[j, pl.ds(g * 16, 16)]
            a16 = a_v[j, pl.ds(g * 16, 16)]
            rs16 = jnp.zeros((16,), jnp.float32)
            for e in range(16):
                rv = rs_v[pl.ds(dst16[e], 16)][0]
                rs16 = jnp.where(lane == e, jnp.full((16,), rv), rs16)
            a_v[j, pl.ds(g * 16, 16)] = a16 * rs16
        return 0
    lax.fori_loop(0, NCHUNK, alpha_body, 0)

    def seg_body(seg, _):
        base = seg * SEGR

        def zb(t, _):
            pltpu.sync_copy(z_v, ash.at[pl.ds(sid * RPS + t * 16, 16)])
            return 0
        lax.fori_loop(0, RPS // 16, zb, 0)
        plsc.subcore_barrier()

        def scale_scatter(j, vr):
            for g in range(CH // 16):
                d16 = dst_v[j, pl.ds(g * 16, 16)] - base
                inb = jnp.logical_and(d16 >= 0, d16 < SEGR)
                idx_c[pl.ds(g * 16, 16)] = jnp.where(inb, d16, TRASH)
            for g in range(CH // 16):
                alpha16 = a_v[j, pl.ds(g * 16, 16)]
                for e in range(16):
                    row = g * 16 + e
                    asp = jnp.full((16,), alpha16[e])
                    for dd in range(D // 16):
                        ix = pl.ds(dd * 16, 16)
                        vr[row, ix] = vr[row, ix] * asp
            pltpu.sync_copy(vr, ash.at[idx_c], add=True)

        def pair_body(p, _):
            ja = 2 * p
            jb = ja + 1
            cva = pltpu.async_copy(v_hbm.at[src_v.at[ja]], vra, sem_va)
            cvb = pltpu.async_copy(v_hbm.at[src_v.at[jb]], vrb, sem_vb)
            cva.wait()
            scale_scatter(ja, vra)
            cvb.wait()
            scale_scatter(jb, vrb)
            return 0

        lax.fori_loop(0, NCHUNK // 2, pair_body, 0)
        jt = NCHUNK - 1
        cv = pltpu.async_copy(v_hbm.at[src_v.at[jt]], vra, sem_va)
        cv.wait()
        scale_scatter(jt, vra)
        plsc.subcore_barrier()

        def db(t, _):
            r0 = sid * RPS + t * 16
            pltpu.sync_copy(ash.at[pl.ds(r0, 16)],
                            agg_hbm.at[cid, pl.ds(base + r0, 16)])
            return 0
        lax.fori_loop(0, RPS // 16, db, 0)
        plsc.subcore_barrier()
        return 0

    lax.fori_loop(0, NSEG, seg_body, 0)


_BLK = 2000


def _proj_body(x_ref, w_ref, q_ref, k_ref, v_ref, s_ref):
    p = jnp.dot(x_ref[...], w_ref[...], preferred_element_type=jnp.float32)
    q_ref[...] = p[:, 0:D]
    k_ref[...] = p[:, D:2 * D]
    v_ref[...] = p[:, 2 * D:3 * D]
    s_ref[...] = p[:, 3 * D:4 * D]


def _proj4(x, wcat):
    spec = pl.BlockSpec((_BLK, D), lambda i: (i, 0))
    return pl.pallas_call(
        _proj_body,
        grid=(N // _BLK,),
        in_specs=[spec, pl.BlockSpec((D, 4 * D), lambda i: (0, 0))],
        out_specs=[spec] * 4,
        out_shape=[jax.ShapeDtypeStruct((N, D), jnp.float32)] * 4,
    )(x, wcat)


def _rs_body(s0_ref, s1_ref, o_ref):
    o_ref[...] = 1.0 / (s0_ref[...] + s1_ref[...] + 1e-16)


def _recip_s(s0, s1):
    s0r = s0.reshape(SSL // 8, NPAD // (SSL // 8))
    s1r = s1.reshape(SSL // 8, NPAD // (SSL // 8))
    spec = pl.BlockSpec(s0r.shape, lambda: (0, 0))
    out = pl.pallas_call(
        _rs_body,
        in_specs=[spec, spec],
        out_specs=spec,
        out_shape=jax.ShapeDtypeStruct(s0r.shape, jnp.float32),
    )(s0r, s1r)
    return out.reshape(NPAD)


def _asm_body(sx_ref, a0_ref, a1_ref, f_ref, o_ref):
    h = sx_ref[...] + a0_ref[...][0] + a1_ref[...][0]
    o_ref[...] = jnp.maximum(h, f_ref[...][0, 0] * h)


def _assemble(sx, aggp, flag):
    spec = pl.BlockSpec((_BLK, D), lambda i: (i, 0))
    return pl.pallas_call(
        _asm_body,
        grid=(N // _BLK,),
        in_specs=[spec,
                  pl.BlockSpec((1, _BLK, D), lambda i: (0, i, 0)),
                  pl.BlockSpec((1, _BLK, D), lambda i: (1, i, 0)),
                  pl.BlockSpec((1, 1), lambda i: (0, 0))],
        out_specs=spec,
        out_shape=jax.ShapeDtypeStruct((N, D), jnp.float32),
    )(sx, aggp, aggp, flag)


def _mm_body(x_ref, w_ref, o_ref):
    o_ref[...] = jnp.dot(x_ref[...], w_ref[...], preferred_element_type=jnp.float32)


def _mm(x, w):
    spec = pl.BlockSpec((_BLK, D), lambda i: (i, 0))
    return pl.pallas_call(
        _mm_body,
        grid=(N // _BLK,),
        in_specs=[spec, pl.BlockSpec((D, D), lambda i: (0, 0))],
        out_specs=spec,
        out_shape=jax.ShapeDtypeStruct((N, D), jnp.float32),
    )(x, w)


def kernel(x, edge_index, graph_len, Wq1, Wk1, Wv1, Ws1, Wq2, Wk2, Wv2, Ws2, Wp):
    src3 = jnp.pad(edge_index[0].reshape(NW, NCHUNK, CH),
                   ((0, 0), (0, NCHP - NCHUNK), (0, 0)))
    dst3 = jnp.pad(edge_index[1].reshape(NW, NCHUNK, CH),
                   ((0, 0), (0, NCHP - NCHUNK), (0, 0)))
    wcats = jnp.stack([
        jnp.concatenate([Wq1, Wk1, Wv1, Ws1], axis=1),
        jnp.concatenate([Wq2, Wk2, Wv2, Ws2], axis=1),
    ])
    flags = jnp.array([0.0, 1.0], jnp.float32).reshape(2, 1, 1)

    def step(h, xs):
        wcat, flag = xs
        q, k, v, sx = _proj4(h, wcat)
        a, s0, s1 = _sc_edge_logits(q, k, src3, dst3)
        rs = _recip_s(s0, s1)
        aggp = _sc_aggregate(v, rs, src3, dst3, a)
        return _assemble(sx, aggp, flag), None

    h2, _ = lax.scan(step, x, (wcats, flags))
    return (h2, _mm(h2, Wp))


# speedup vs baseline: 1.0823x; 1.0823x over previous
import functools
import math

import jax
import jax.numpy as jnp
from jax import lax
from jax.experimental import pallas as pl
from jax.experimental.pallas import tpu as pltpu
from jax.experimental.pallas import tpu_sc as plsc

N = 10000
E = 320000
D = 128
NC = 2
NS = 16
NW = NC * NS
EPW = E // NW
CH = 80
NCHUNK = EPW // CH
NCHP = 128
NPAD = 10240
SSL = NPAD // NS
INV_SQRT_D = 1.0 / math.sqrt(float(D))

_mesh = plsc.VectorSubcoreMesh(core_axis_name="c", subcore_axis_name="s")
_GD = lax.GatherDimensionNumbers(
    offset_dims=(), collapsed_slice_dims=(0,), start_index_map=(0,))


def _lane_shuffle(v, idx):
    return lax.gather(v, idx.reshape(16, 1), dimension_numbers=_GD,
                      slice_sizes=(1,), mode=lax.GatherScatterMode.PROMISE_IN_BOUNDS)


@functools.partial(
    pl.kernel,
    out_type=(
        jax.ShapeDtypeStruct((NW, NCHP, CH), jnp.float32),
        jax.ShapeDtypeStruct((NPAD,), jnp.float32),
        jax.ShapeDtypeStruct((NPAD,), jnp.float32),
    ),
    mesh=_mesh,
    scratch_types=[
        pltpu.VMEM((NCHP, CH), jnp.int32),
        pltpu.VMEM((NCHP, CH), jnp.int32),
        pltpu.VMEM((NCHP, CH), jnp.float32),
        pltpu.VMEM((CH, D), jnp.float32),
        pltpu.VMEM((CH, D), jnp.float32),
        pltpu.VMEM((SSL,), jnp.float32),
        pltpu.VMEM_SHARED((NPAD,), jnp.float32),
        pltpu.SemaphoreType.DMA,
        pltpu.SemaphoreType.DMA,
    ],
)
def _sc_edge_logits(q_hbm, k_hbm, src_hbm, dst_hbm, a_hbm, s0_hbm, s1_hbm,
                    src_v, dst_v, a_v, qrows, krows, z_v, ssh, sem_q, sem_k):
    cid = lax.axis_index("c")
    sid = lax.axis_index("s")
    wid = sid * NC + cid
    lane = lax.iota(jnp.int32, 16)

    def zbody(i, _):
        z_v[pl.ds(i * 16, 16)] = jnp.zeros((16,), jnp.float32)
        return 0
    lax.fori_loop(0, SSL // 16, zbody, 0)
    pltpu.sync_copy(z_v, ssh.at[pl.ds(sid * SSL, SSL)])
    pltpu.sync_copy(src_hbm.at[wid], src_v)
    pltpu.sync_copy(dst_hbm.at[wid], dst_v)
    plsc.subcore_barrier()

    def chunk_body(j, _):
        cq = pltpu.async_copy(q_hbm.at[dst_v.at[j]], qrows, sem_q)
        ck = pltpu.async_copy(k_hbm.at[src_v.at[j]], krows, sem_k)
        cq.wait()
        ck.wait()
        for g in range(CH // 16):
            logits = jnp.zeros((16,), jnp.float32)
            for e in range(16):
                row = g * 16 + e

                def dbody(dd, acc):
                    ix = pl.ds(dd * 16, 16)
                    return acc + qrows[row, ix] * krows[row, ix]
                acc = lax.fori_loop(0, D // 16, dbody,
                                    jnp.zeros((16,), jnp.float32), unroll=8)
                for sh in (8, 4, 2, 1):
                    acc = acc + _lane_shuffle(acc, jnp.bitwise_xor(lane, sh))
                logits = jnp.where(lane == e, acc, logits)
            a_v[j, pl.ds(g * 16, 16)] = jnp.exp(logits * INV_SQRT_D)
        pltpu.sync_copy(a_v.at[j], ssh.at[dst_v.at[j]], add=True)
        return 0

    lax.fori_loop(0, NCHUNK, chunk_body, 0)
    pltpu.sync_copy(a_v, a_hbm.at[wid])
    plsc.subcore_barrier()
    pltpu.sync_copy(ssh.at[pl.ds(sid * SSL, SSL)], z_v)

    @pl.when(cid == 0)
    def _():
        pltpu.sync_copy(z_v, s0_hbm.at[pl.ds(sid * SSL, SSL)])

    @pl.when(cid == 1)
    def _():
        pltpu.sync_copy(z_v, s1_hbm.at[pl.ds(sid * SSL, SSL)])


NSEG = 2
SEGR = NPAD // NSEG
ASHR = SEGR + 16
RPS = SEGR // NS


@functools.partial(
    pl.kernel,
    out_type=jax.ShapeDtypeStruct((NC, NPAD, D), jnp.float32),
    mesh=_mesh,
    scratch_types=[
        pltpu.VMEM((NCHP, CH), jnp.int32),
        pltpu.VMEM((NCHP, CH), jnp.int32),
        pltpu.VMEM((NCHP, CH), jnp.float32),
        pltpu.VMEM((NCHP, CH), jnp.int32),
        pltpu.VMEM((NPAD + 16,), jnp.float32),
        pltpu.VMEM((CH, D), jnp.float32),
        pltpu.VMEM((16, D), jnp.float32),
        pltpu.VMEM_SHARED((ASHR, D), jnp.float32),
        pltpu.SemaphoreType.DMA,
    ],
)
def _sc_aggregate(v_hbm, rs_hbm, src_hbm, dst_hbm, a_hbm, agg_hbm,
                  src_v, dst_v, a_v, rd_v, rs_v, vrows, z_v, ash, sem_v):
    cid = lax.axis_index("c")
    sid = lax.axis_index("s")
    wid = sid * NC + cid
    lane = lax.iota(jnp.int32, 16)

    def zbody(i, _):
        r = i // (D // 16)
        col = (i % (D // 16)) * 16
        z_v[r, pl.ds(col, 16)] = jnp.zeros((16,), jnp.float32)
        return 0
    lax.fori_loop(0, 16 * (D // 16), zbody, 0)

    pltpu.sync_copy(rs_hbm, rs_v.at[pl.ds(0, NPAD)])
    pltpu.sync_copy(src_hbm.at[wid], src_v)
    pltpu.sync_copy(dst_hbm.at[wid], dst_v)
    pltpu.sync_copy(a_hbm.at[wid], a_v)

    def al_body(j, _):
        for g in range(CH // 16):
            gix = pl.ds(g * 16, 16)
            dst16 = dst_v[j, gix]
            rs16 = jnp.zeros((16,), jnp.float32)
            for e in range(16):
                rv = rs_v[pl.ds(dst16[e], 16)][0]
                rs16 = jnp.where(lane == e, jnp.full((16,), rv), rs16)
            a_v[j, gix] = a_v[j, gix] * rs16
        return 0
    lax.fori_loop(0, NCHUNK, al_body, 0)

    def seg_body(seg, _):
        dbase = seg * SEGR

        def zb(t, _):
            pltpu.sync_copy(z_v, ash.at[pl.ds(sid * RPS + t * 16, 16)])
            return 0
        lax.fori_loop(0, RPS // 16, zb, 0)

        def rm_body(j, _):
            for g in range(CH // 16):
                gix = pl.ds(g * 16, 16)
                d16 = dst_v[j, gix] - dbase
                in_seg = (d16 >= 0) & (d16 < SEGR)
                rd_v[j, gix] = jnp.where(in_seg, d16, SEGR)
            return 0
        lax.fori_loop(0, NCHUNK, rm_body, 0)
        plsc.subcore_barrier()

        def chunk_body(j, _):
            cp = pltpu.async_copy(v_hbm.at[src_v.at[j]], vrows, sem_v)
            cp.wait()
            for g in range(CH // 16):
                al16 = a_v[j, pl.ds(g * 16, 16)]
                for e in range(16):
                    row = g * 16 + e
                    asp = jnp.full((16,), al16[e])
                    for dd in range(D // 16):
                        ix = pl.ds(dd * 16, 16)
                        vrows[row, ix] = vrows[row, ix] * asp
            pltpu.sync_copy(vrows, ash.at[rd_v.at[j]], add=True)
            return 0

        lax.fori_loop(0, NCHUNK, chunk_body, 0)
        plsc.subcore_barrier()

        def db(t, _):
            r0 = sid * RPS + t * 16
            pltpu.sync_copy(ash.at[pl.ds(r0, 16)],
                            agg_hbm.at[cid, pl.ds(dbase + r0, 16)])
            return 0
        lax.fori_loop(0, RPS // 16, db, 0)
        plsc.subcore_barrier()
        return 0

    lax.fori_loop(0, NSEG, seg_body, 0)


_BLK = 2000


def _proj_body(x_ref, w_ref, q_ref, k_ref, v_ref, s_ref):
    p = jnp.dot(x_ref[...], w_ref[...], preferred_element_type=jnp.float32)
    q_ref[...] = p[:, 0:D]
    k_ref[...] = p[:, D:2 * D]
    v_ref[...] = p[:, 2 * D:3 * D]
    s_ref[...] = p[:, 3 * D:4 * D]


def _proj4(x, wcat):
    spec = pl.BlockSpec((_BLK, D), lambda i: (i, 0))
    return pl.pallas_call(
        _proj_body,
        grid=(N // _BLK,),
        in_specs=[spec, pl.BlockSpec((D, 4 * D), lambda i: (0, 0))],
        out_specs=[spec] * 4,
        out_shape=[jax.ShapeDtypeStruct((N, D), jnp.float32)] * 4,
    )(x, wcat)


def _rs_body(s0_ref, s1_ref, o_ref):
    o_ref[...] = 1.0 / (s0_ref[...] + s1_ref[...] + 1e-16)


def _recip_s(s0, s1):
    s0r = s0.reshape(SSL // 8, NPAD // (SSL // 8))
    s1r = s1.reshape(SSL // 8, NPAD // (SSL // 8))
    spec = pl.BlockSpec(s0r.shape, lambda: (0, 0))
    out = pl.pallas_call(
        _rs_body,
        in_specs=[spec, spec],
        out_specs=spec,
        out_shape=jax.ShapeDtypeStruct(s0r.shape, jnp.float32),
    )(s0r, s1r)
    return out.reshape(NPAD)


def _asm_body(sx_ref, a0_ref, a1_ref, f_ref, o_ref):
    h = sx_ref[...] + a0_ref[...][0] + a1_ref[...][0]
    o_ref[...] = jnp.maximum(h, f_ref[...][0, 0] * h)


def _assemble(sx, aggp, flag):
    spec = pl.BlockSpec((_BLK, D), lambda i: (i, 0))
    return pl.pallas_call(
        _asm_body,
        grid=(N // _BLK,),
        in_specs=[spec,
                  pl.BlockSpec((1, _BLK, D), lambda i: (0, i, 0)),
                  pl.BlockSpec((1, _BLK, D), lambda i: (1, i, 0)),
                  pl.BlockSpec((1, 1), lambda i: (0, 0))],
        out_specs=spec,
        out_shape=jax.ShapeDtypeStruct((N, D), jnp.float32),
    )(sx, aggp, aggp, flag)


def _mm_body(x_ref, w_ref, o_ref):
    o_ref[...] = jnp.dot(x_ref[...], w_ref[...], preferred_element_type=jnp.float32)


def _mm(x, w):
    spec = pl.BlockSpec((_BLK, D), lambda i: (i, 0))
    return pl.pallas_call(
        _mm_body,
        grid=(N // _BLK,),
        in_specs=[spec, pl.BlockSpec((D, D), lambda i: (0, 0))],
        out_specs=spec,
        out_shape=jax.ShapeDtypeStruct((N, D), jnp.float32),
    )(x, w)


def kernel(x, edge_index, graph_len, Wq1, Wk1, Wv1, Ws1, Wq2, Wk2, Wv2, Ws2, Wp):
    src3 = jnp.pad(edge_index[0].reshape(NW, NCHUNK, CH),
                   ((0, 0), (0, NCHP - NCHUNK), (0, 0)))
    dst3 = jnp.pad(edge_index[1].reshape(NW, NCHUNK, CH),
                   ((0, 0), (0, NCHP - NCHUNK), (0, 0)))
    wcats = jnp.stack([
        jnp.concatenate([Wq1, Wk1, Wv1, Ws1], axis=1),
        jnp.concatenate([Wq2, Wk2, Wv2, Ws2], axis=1),
    ])
    flags = jnp.array([0.0, 1.0], jnp.float32).reshape(2, 1, 1)

    def step(h, xs):
        wcat, flag = xs
        q, k, v, sx = _proj4(h, wcat)
        a, s0, s1 = _sc_edge_logits(q, k, src3, dst3)
        rs = _recip_s(s0, s1)
        aggp = _sc_aggregate(v, rs, src3, dst3, a)
        return _assemble(sx, aggp, flag), None

    h2, _ = lax.scan(step, x, (wcats, flags))
    return (h2, _mm(h2, Wp))


# speedup vs baseline: 1.4263x; 1.3178x over previous
import functools
import math

import jax
import jax.numpy as jnp
from jax import lax
from jax.experimental import pallas as pl
from jax.experimental.pallas import tpu as pltpu
from jax.experimental.pallas import tpu_sc as plsc

N = 10000
E = 320000
D = 128
NC = 2
NS = 16
NW = NC * NS
EPW = E // NW
CH = 80
NCHUNK = EPW // CH
NCHP = 128
NPAD = 10240
SSL = NPAD // NS
INV_SQRT_D = 1.0 / math.sqrt(float(D))

_mesh = plsc.VectorSubcoreMesh(core_axis_name="c", subcore_axis_name="s")
_GD = lax.GatherDimensionNumbers(
    offset_dims=(), collapsed_slice_dims=(0,), start_index_map=(0,))


def _lane_shuffle(v, idx):
    return lax.gather(v, idx.reshape(16, 1), dimension_numbers=_GD,
                      slice_sizes=(1,), mode=lax.GatherScatterMode.PROMISE_IN_BOUNDS)


@functools.partial(
    pl.kernel,
    out_type=(
        jax.ShapeDtypeStruct((NW, NCHP, CH), jnp.float32),
        jax.ShapeDtypeStruct((NPAD,), jnp.float32),
        jax.ShapeDtypeStruct((NPAD,), jnp.float32),
    ),
    mesh=_mesh,
    scratch_types=[
        pltpu.VMEM((NCHP, CH), jnp.int32),
        pltpu.VMEM((NCHP, CH), jnp.int32),
        pltpu.VMEM((NCHP, CH), jnp.float32),
        pltpu.VMEM((CH, D), jnp.float32),
        pltpu.VMEM((CH, D), jnp.float32),
        pltpu.VMEM((CH, D), jnp.float32),
        pltpu.VMEM((CH, D), jnp.float32),
        pltpu.VMEM((SSL,), jnp.float32),
        pltpu.VMEM_SHARED((NPAD,), jnp.float32),
        pltpu.SemaphoreType.DMA,
        pltpu.SemaphoreType.DMA,
        pltpu.SemaphoreType.DMA,
        pltpu.SemaphoreType.DMA,
    ],
)
def _sc_edge_logits(q_hbm, k_hbm, src_hbm, dst_hbm, a_hbm, s0_hbm, s1_hbm,
                    src_v, dst_v, a_v, qrows, krows, qrows2, krows2, z_v, ssh,
                    sem_q, sem_k, sem_q2, sem_k2):
    cid = lax.axis_index("c")
    sid = lax.axis_index("s")
    wid = sid * NC + cid
    lane = lax.iota(jnp.int32, 16)

    def zbody(i, _):
        z_v[pl.ds(i * 16, 16)] = jnp.zeros((16,), jnp.float32)
        return 0
    lax.fori_loop(0, SSL // 16, zbody, 0)
    pltpu.sync_copy(z_v, ssh.at[pl.ds(sid * SSL, SSL)])
    pltpu.sync_copy(src_hbm.at[wid], src_v)
    pltpu.sync_copy(dst_hbm.at[wid], dst_v)
    plsc.subcore_barrier()

    def _issue(jj, qr, kr, sq, sk):
        pltpu.async_copy(q_hbm.at[dst_v.at[jj]], qr, sq)
        pltpu.async_copy(k_hbm.at[src_v.at[jj]], kr, sk)

    def _drain(jj, qr, kr, sq, sk):
        pltpu.make_async_copy(q_hbm.at[dst_v.at[jj]], qr, sq).wait()
        pltpu.make_async_copy(k_hbm.at[src_v.at[jj]], kr, sk).wait()

    def _compute(j, qr, kr):
        for g in range(CH // 16):
            logits = jnp.zeros((16,), jnp.float32)
            for e in range(16):
                row = g * 16 + e

                def dbody(dd, acc):
                    ix = pl.ds(dd * 16, 16)
                    return acc + qr[row, ix] * kr[row, ix]
                acc = lax.fori_loop(0, D // 16, dbody,
                                    jnp.zeros((16,), jnp.float32), unroll=8)
                for sh in (8, 4, 2, 1):
                    acc = acc + _lane_shuffle(acc, jnp.bitwise_xor(lane, sh))
                logits = jnp.where(lane == e, acc, logits)
            a_v[j, pl.ds(g * 16, 16)] = jnp.exp(logits * INV_SQRT_D)
        pltpu.sync_copy(a_v.at[j], ssh.at[dst_v.at[j]], add=True)

    _issue(0, qrows, krows, sem_q, sem_k)

    def pair_body(p, _):
        j0 = 2 * p
        j1 = j0 + 1
        _drain(j0, qrows, krows, sem_q, sem_k)
        _issue(j1, qrows2, krows2, sem_q2, sem_k2)
        _compute(j0, qrows, krows)
        _drain(j1, qrows2, krows2, sem_q2, sem_k2)
        _issue(j1 + 1, qrows, krows, sem_q, sem_k)
        _compute(j1, qrows2, krows2)
        return 0

    lax.fori_loop(0, (NCHUNK - 1) // 2, pair_body, 0)
    _drain(NCHUNK - 1, qrows, krows, sem_q, sem_k)
    _compute(NCHUNK - 1, qrows, krows)
    pltpu.sync_copy(a_v, a_hbm.at[wid])
    plsc.subcore_barrier()
    pltpu.sync_copy(ssh.at[pl.ds(sid * SSL, SSL)], z_v)

    @pl.when(cid == 0)
    def _():
        pltpu.sync_copy(z_v, s0_hbm.at[pl.ds(sid * SSL, SSL)])

    @pl.when(cid == 1)
    def _():
        pltpu.sync_copy(z_v, s1_hbm.at[pl.ds(sid * SSL, SSL)])


NSEG = 2
SEGR = NPAD // NSEG
ASHR = SEGR + 16
RPS = SEGR // NS


@functools.partial(
    pl.kernel,
    out_type=jax.ShapeDtypeStruct((NC, NPAD, D), jnp.float32),
    mesh=_mesh,
    scratch_types=[
        pltpu.VMEM((NCHP, CH), jnp.int32),
        pltpu.VMEM((NCHP, CH), jnp.int32),
        pltpu.VMEM((NCHP, CH), jnp.float32),
        pltpu.VMEM((8, CH), jnp.int32),
        pltpu.VMEM((NPAD + 16,), jnp.float32),
        pltpu.VMEM((CH, D), jnp.float32),
        pltpu.VMEM((CH, D), jnp.float32),
        pltpu.VMEM((16, D), jnp.float32),
        pltpu.VMEM_SHARED((ASHR, D), jnp.float32),
        pltpu.SemaphoreType.DMA,
        pltpu.SemaphoreType.DMA,
    ],
)
def _sc_aggregate(v_hbm, rs_hbm, src_hbm, dst_hbm, a_hbm, agg_hbm,
                  src_v, dst_v, a_v, rd_v, rs_v, vrows, vrows2, z_v, ash,
                  sem_v, sem_v2):
    cid = lax.axis_index("c")
    sid = lax.axis_index("s")
    wid = sid * NC + cid
    lane = lax.iota(jnp.int32, 16)

    def zbody(i, _):
        r = i // (D // 16)
        col = (i % (D // 16)) * 16
        z_v[r, pl.ds(col, 16)] = jnp.zeros((16,), jnp.float32)
        return 0
    lax.fori_loop(0, 16 * (D // 16), zbody, 0)

    pltpu.sync_copy(rs_hbm, rs_v.at[pl.ds(0, NPAD)])
    pltpu.sync_copy(src_hbm.at[wid], src_v)
    pltpu.sync_copy(dst_hbm.at[wid], dst_v)
    pltpu.sync_copy(a_hbm.at[wid], a_v)

    def al_body(j, _):
        for g in range(CH // 16):
            gix = pl.ds(g * 16, 16)
            dst16 = dst_v[j, gix]
            rs16 = jnp.zeros((16,), jnp.float32)
            for e in range(16):
                rv = rs_v[pl.ds(dst16[e], 16)][0]
                rs16 = jnp.where(lane == e, jnp.full((16,), rv), rs16)
            a_v[j, gix] = a_v[j, gix] * rs16
        return 0
    lax.fori_loop(0, NCHUNK, al_body, 0)

    def seg_body(seg, _):
        dbase = seg * SEGR

        def zb(t, _):
            pltpu.sync_copy(z_v, ash.at[pl.ds(sid * RPS + t * 16, 16)])
            return 0
        lax.fori_loop(0, RPS // 16, zb, 0)
        plsc.subcore_barrier()

        def _issue_v(jj, vr, sv):
            pltpu.async_copy(v_hbm.at[src_v.at[jj]], vr, sv)

        def _drain_v(jj, vr, sv):
            pltpu.make_async_copy(v_hbm.at[src_v.at[jj]], vr, sv).wait()

        def _compute_v(j, vr):
            for g in range(CH // 16):
                gix = pl.ds(g * 16, 16)
                d16 = dst_v[j, gix] - dbase
                in_seg = (d16 >= 0) & (d16 < SEGR)
                rd_v[0, gix] = jnp.where(in_seg, d16, SEGR)
                al16 = a_v[j, gix]
                for e in range(16):
                    row = g * 16 + e
                    asp = jnp.full((16,), al16[e])
                    for dd in range(D // 16):
                        ix = pl.ds(dd * 16, 16)
                        vr[row, ix] = vr[row, ix] * asp
            pltpu.sync_copy(vr, ash.at[rd_v.at[0]], add=True)

        _issue_v(0, vrows, sem_v)

        def vpair_body(p, _):
            j0 = 2 * p
            j1 = j0 + 1
            _drain_v(j0, vrows, sem_v)
            _issue_v(j1, vrows2, sem_v2)
            _compute_v(j0, vrows)
            _drain_v(j1, vrows2, sem_v2)
            _issue_v(j1 + 1, vrows, sem_v)
            _compute_v(j1, vrows2)
            return 0

        lax.fori_loop(0, (NCHUNK - 1) // 2, vpair_body, 0)
        _drain_v(NCHUNK - 1, vrows, sem_v)
        _compute_v(NCHUNK - 1, vrows)
        plsc.subcore_barrier()

        def db(t, _):
            r0 = sid * RPS + t * 16
            pltpu.sync_copy(ash.at[pl.ds(r0, 16)],
                            agg_hbm.at[cid, pl.ds(dbase + r0, 16)])
            return 0
        lax.fori_loop(0, RPS // 16, db, 0)
        plsc.subcore_barrier()
        return 0

    lax.fori_loop(0, NSEG, seg_body, 0)


_BLK = 2000


def _proj_body(x_ref, w_ref, q_ref, k_ref, v_ref, s_ref):
    p = jnp.dot(x_ref[...], w_ref[...], preferred_element_type=jnp.float32)
    q_ref[...] = p[:, 0:D]
    k_ref[...] = p[:, D:2 * D]
    v_ref[...] = p[:, 2 * D:3 * D]
    s_ref[...] = p[:, 3 * D:4 * D]


def _proj4(x, wcat):
    spec = pl.BlockSpec((_BLK, D), lambda i: (i, 0))
    return pl.pallas_call(
        _proj_body,
        grid=(N // _BLK,),
        in_specs=[spec, pl.BlockSpec((D, 4 * D), lambda i: (0, 0))],
        out_specs=[spec] * 4,
        out_shape=[jax.ShapeDtypeStruct((N, D), jnp.float32)] * 4,
    )(x, wcat)


def _rs_body(s0_ref, s1_ref, o_ref):
    o_ref[...] = 1.0 / (s0_ref[...] + s1_ref[...] + 1e-16)


def _recip_s(s0, s1):
    s0r = s0.reshape(SSL // 8, NPAD // (SSL // 8))
    s1r = s1.reshape(SSL // 8, NPAD // (SSL // 8))
    spec = pl.BlockSpec(s0r.shape, lambda: (0, 0))
    out = pl.pallas_call(
        _rs_body,
        in_specs=[spec, spec],
        out_specs=spec,
        out_shape=jax.ShapeDtypeStruct(s0r.shape, jnp.float32),
    )(s0r, s1r)
    return out.reshape(NPAD)


def _asm_body(sx_ref, a0_ref, a1_ref, f_ref, o_ref):
    h = sx_ref[...] + a0_ref[...][0] + a1_ref[...][0]
    o_ref[...] = jnp.maximum(h, f_ref[...][0, 0] * h)


def _assemble(sx, aggp, flag):
    spec = pl.BlockSpec((_BLK, D), lambda i: (i, 0))
    return pl.pallas_call(
        _asm_body,
        grid=(N // _BLK,),
        in_specs=[spec,
                  pl.BlockSpec((1, _BLK, D), lambda i: (0, i, 0)),
                  pl.BlockSpec((1, _BLK, D), lambda i: (1, i, 0)),
                  pl.BlockSpec((1, 1), lambda i: (0, 0))],
        out_specs=spec,
        out_shape=jax.ShapeDtypeStruct((N, D), jnp.float32),
    )(sx, aggp, aggp, flag)


def _mm_body(x_ref, w_ref, o_ref):
    o_ref[...] = jnp.dot(x_ref[...], w_ref[...], preferred_element_type=jnp.float32)


def _mm(x, w):
    spec = pl.BlockSpec((_BLK, D), lambda i: (i, 0))
    return pl.pallas_call(
        _mm_body,
        grid=(N // _BLK,),
        in_specs=[spec, pl.BlockSpec((D, D), lambda i: (0, 0))],
        out_specs=spec,
        out_shape=jax.ShapeDtypeStruct((N, D), jnp.float32),
    )(x, w)


def kernel(x, edge_index, graph_len, Wq1, Wk1, Wv1, Ws1, Wq2, Wk2, Wv2, Ws2, Wp):
    src3 = jnp.pad(edge_index[0].reshape(NW, NCHUNK, CH),
                   ((0, 0), (0, NCHP - NCHUNK), (0, 0)))
    dst3 = jnp.pad(edge_index[1].reshape(NW, NCHUNK, CH),
                   ((0, 0), (0, NCHP - NCHUNK), (0, 0)))
    wcats = jnp.stack([
        jnp.concatenate([Wq1, Wk1, Wv1, Ws1], axis=1),
        jnp.concatenate([Wq2, Wk2, Wv2, Ws2], axis=1),
    ])
    flags = jnp.array([0.0, 1.0], jnp.float32).reshape(2, 1, 1)

    def step(h, xs):
        wcat, flag = xs
        q, k, v, sx = _proj4(h, wcat)
        a, s0, s1 = _sc_edge_logits(q, k, src3, dst3)
        rs = _recip_s(s0, s1)
        aggp = _sc_aggregate(v, rs, src3, dst3, a)
        return _assemble(sx, aggp, flag), None

    h2, _ = lax.scan(step, x, (wcats, flags))
    return (h2, _mm(h2, Wp))
